# baseline (speedup 1.0000x reference)
import functools

import jax
import jax.numpy as jnp
from jax import lax
from jax.experimental import pallas as pl
from jax.experimental.pallas import tpu as pltpu
from jax.experimental.pallas import tpu_sc as plsc

CHUNK = 128
NBUF = 8


@functools.lru_cache(maxsize=None)
def _make_lookup(n_rows, dim):
    info = plsc.get_sparse_core_info()
    nc, ns = info.num_cores, info.num_subcores
    nw = nc * ns
    assert n_rows % (nw * CHUNK) == 0, (n_rows, nw, CHUNK)
    chunks_per_w = n_rows // (nw * CHUNK)
    assert chunks_per_w % NBUF == 0, (chunks_per_w, NBUF)
    n_rounds = chunks_per_w // NBUF
    rows_per_w = chunks_per_w * CHUNK
    mesh = plsc.VectorSubcoreMesh(core_axis_name="c", subcore_axis_name="s")

    @functools.partial(
        pl.kernel,
        mesh=mesh,
        out_type=jax.ShapeDtypeStruct((n_rows, dim), jnp.float32),
        scratch_types=[
            pltpu.VMEM((chunks_per_w, CHUNK), jnp.int32),
            pltpu.VMEM((NBUF, CHUNK, dim), jnp.float32),
            pltpu.SemaphoreType.DMA,
            pltpu.SemaphoreType.DMA,
        ],
        compiler_params=pltpu.CompilerParams(use_tc_tiling_on_sc=False),
    )
    def lookup(emb_hbm, idx_hbm, out_hbm, idx_v, rows_v, gsem, ssem):
        wid = lax.axis_index("s") * nc + lax.axis_index("c")
        row0 = wid * rows_per_w
        pltpu.sync_copy(idx_hbm.at[pl.ds(wid * chunks_per_w, chunks_per_w)], idx_v)

        def gather(c, b):
            return pltpu.make_async_copy(emb_hbm.at[idx_v.at[c]], rows_v.at[b], gsem)

        def store(c, b):
            return pltpu.make_async_copy(
                rows_v.at[b], out_hbm.at[pl.ds(row0 + c * CHUNK, CHUNK)], ssem)

        for b in range(NBUF):
            gather(b, b).start()

        def round_body(r, carry):
            c0 = r * NBUF
            for b in range(NBUF):
                gather(c0 + b, b).wait()
            for b in range(NBUF):
                store(c0 + b, b).start()
            for b in range(NBUF):
                store(c0 + b, b).wait()
            for b in range(NBUF):
                gather(c0 + NBUF + b, b).start()
            return carry

        lax.fori_loop(0, n_rounds - 1, round_body, 0)

        c0 = (n_rounds - 1) * NBUF
        for b in range(NBUF):
            gather(c0 + b, b).wait()
        for b in range(NBUF):
            store(c0 + b, b).start()
        for b in range(NBUF):
            store(c0 + b, b).wait()

    return lookup


def kernel(item_seqs, emb):
    bsz, seq = item_seqs.shape
    _, dim = emb.shape
    n_rows = bsz * seq
    idx = item_seqs.reshape(n_rows // CHUNK, CHUNK)
    out = _make_lookup(n_rows, dim)(emb, idx)
    return out.reshape(bsz, seq, dim)
